# Initial kernel scaffold; baseline (speedup 1.0000x reference)
import jax
import jax.numpy as jnp
from jax.experimental import pallas as pl


def kernel(x, edge_index, edge_attr, eigvecs, eigvals, eps_param, phi_W1, phi_b1, phi_W2, phi_b2, edge_W, edge_b, g0_We, g0_be, g0_W1, g0_b1, g0_W2, g0_b2, g0_eps, g1_We, g1_be, g1_W1, g1_b1, g1_W2, g1_b2, g1_eps, out_W, out_b):
    raise NotImplementedError("write your pallas kernel here")


# speedup vs baseline: 2.6846x; 2.6846x over previous
import functools

import jax
import jax.numpy as jnp
from jax import lax
from jax.experimental import pallas as pl
from jax.experimental.pallas import tpu as pltpu
from jax.experimental.pallas import tpu_sc as plsc

N = 10000
E = 320000
H = 128
NV = 16
DE = 16
EMB = 64
PE = 32
HID = 2 * H

F32 = jnp.float32


_BN = 1000


def _eigen_body(x_ref, vec_ref, val_ref, eps_ref, w1_ref, b1_ref, w2_ref,
                b2_ref, out_ref):
  ev = val_ref[...] + eps_ref[...]
  ev = jnp.where(jnp.isnan(ev), 0.0, ev)
  vec = jnp.where(jnp.isnan(vec_ref[...]), 0.0, vec_ref[...])
  w1 = w1_ref[...]
  b1 = b1_ref[...]
  w2 = w2_ref[...]
  b2 = b2_ref[...]
  acc = jnp.zeros((_BN, H), F32)
  for v in range(NV):
    t1 = vec[:, v:v + 1] * w1[0:1, :] + ev[:, v:v + 1] * w1[1:2, :] + b1
    t1 = jnp.maximum(t1, 0.0)
    t2 = jnp.dot(t1, w2, preferred_element_type=F32) + b2
    acc = acc + jnp.maximum(t2, 0.0)
  out_ref[...] = x_ref[...] + acc


def _eigen_stage(x, eigvecs, eigvals, eps_row, phi_W1, phi_b1, phi_W2,
                 phi_b2):
  grid = N // _BN
  return pl.pallas_call(
      _eigen_body,
      grid=(grid,),
      in_specs=[
          pl.BlockSpec((_BN, H), lambda i: (i, 0)),
          pl.BlockSpec((_BN, NV), lambda i: (i, 0)),
          pl.BlockSpec((_BN, NV), lambda i: (i, 0)),
          pl.BlockSpec((1, NV), lambda i: (0, 0)),
          pl.BlockSpec((2, HID), lambda i: (0, 0)),
          pl.BlockSpec((1, HID), lambda i: (0, 0)),
          pl.BlockSpec((HID, H), lambda i: (0, 0)),
          pl.BlockSpec((1, H), lambda i: (0, 0)),
      ],
      out_specs=pl.BlockSpec((_BN, H), lambda i: (i, 0)),
      out_shape=jax.ShapeDtypeStruct((N, H), F32),
  )(x, eigvecs, eigvals, eps_row, phi_W1, phi_b1, phi_W2, phi_b2)


_BE = 4000


def _edge_body(ea_ref, C0_ref, c0_ref, C1_ref, c1_ref, e0_ref, e1_ref):
  ea = ea_ref[...]
  e0_ref[...] = jnp.dot(ea, C0_ref[...], preferred_element_type=F32) + c0_ref[...]
  e1_ref[...] = jnp.dot(ea, C1_ref[...], preferred_element_type=F32) + c1_ref[...]


def _edge_stage(edge_attr, C0, c0_row, C1, c1_row):
  grid = E // _BE
  return pl.pallas_call(
      _edge_body,
      grid=(grid,),
      in_specs=[
          pl.BlockSpec((_BE, DE), lambda i: (i, 0)),
          pl.BlockSpec((DE, H), lambda i: (0, 0)),
          pl.BlockSpec((1, H), lambda i: (0, 0)),
          pl.BlockSpec((DE, EMB), lambda i: (0, 0)),
          pl.BlockSpec((1, EMB), lambda i: (0, 0)),
      ],
      out_specs=[
          pl.BlockSpec((_BE, H), lambda i: (i, 0)),
          pl.BlockSpec((_BE, EMB), lambda i: (i, 0)),
      ],
      out_shape=[
          jax.ShapeDtypeStruct((E, H), F32),
          jax.ShapeDtypeStruct((E, EMB), F32),
      ],
  )(edge_attr, C0, c0_row, C1, c1_row)


_NC = 2
_NS = 16
_C = 80
_ZR = 128
_NP = 10240


def _make_sc_segment(D):
  per_tile = E // (_NC * _NS)
  chunks = per_tile // _C
  rp = _NP // _NS

  mesh = plsc.VectorSubcoreMesh(core_axis_name="c", subcore_axis_name="s",
                                num_cores=_NC, num_subcores=_NS)

  @functools.partial(
      pl.kernel,
      out_type=jax.ShapeDtypeStruct((_NC, _NP, D), F32),
      mesh=mesh,
      scratch_types=[
          pltpu.VMEM((_C,), jnp.int32),
          pltpu.VMEM((_C,), jnp.int32),
          pltpu.VMEM((_C, D), F32),
          pltpu.VMEM((_ZR, D), F32),
          pltpu.VMEM_SHARED((_NP, D), F32),
          pltpu.SemaphoreType.DMA,
      ],
      compiler_params=pltpu.CompilerParams(use_tc_tiling_on_sc=False),
  )
  def seg(h_hbm, e_hbm, src_hbm, dst_hbm, out_hbm, sidx, didx, buf, zbuf,
          agg, sem):
    c = lax.axis_index("c")
    s = lax.axis_index("s")
    wid = c * _NS + s

    def zrow(i, carry):
      for j in range(D // 16):
        zbuf[i, pl.ds(16 * j, 16)] = jnp.zeros((16,), F32)
      return carry

    lax.fori_loop(0, _ZR, zrow, 0)
    for q in range(rp // _ZR):
      pltpu.sync_copy(zbuf, agg.at[pl.ds(s * rp + q * _ZR, _ZR)])
    plsc.subcore_barrier()

    base0 = wid * per_tile

    def chunk(k, carry):
      b = base0 + k * _C
      pltpu.sync_copy(src_hbm.at[pl.ds(b, _C)], sidx)
      pltpu.sync_copy(dst_hbm.at[pl.ds(b, _C)], didx)
      pltpu.sync_copy(e_hbm.at[pl.ds(b, _C)], buf)
      pltpu.async_copy(h_hbm.at[sidx], buf, sem, add=True).wait()

      def rrow(i, cc):
        for j in range(D // 16):
          sl = pl.ds(16 * j, 16)
          buf[i, sl] = jnp.maximum(buf[i, sl], 0.0)
        return cc

      lax.fori_loop(0, _C, rrow, 0)
      pltpu.sync_copy(buf, agg.at[didx], add=True)
      return carry

    lax.fori_loop(0, chunks, chunk, 0)
    plsc.subcore_barrier()
    pltpu.sync_copy(agg.at[pl.ds(s * rp, rp)],
                    out_hbm.at[c, pl.ds(s * rp, rp)])

  return seg


_sc_cache = {}


def _sc_segment(D):
  if D not in _sc_cache:
    _sc_cache[D] = _make_sc_segment(D)
  return _sc_cache[D]


def _node0_body(h_ref, p0_ref, p1_ref, sc_ref, w1_ref, b1_ref, w2_ref,
                b2_ref, out_ref):
  hv = sc_ref[0, 0] * h_ref[...] + p0_ref[0] + p1_ref[0]
  t = jnp.maximum(jnp.dot(hv, w1_ref[...], preferred_element_type=F32)
                  + b1_ref[...], 0.0)
  out_ref[...] = jnp.maximum(
      jnp.dot(t, w2_ref[...], preferred_element_type=F32) + b2_ref[...], 0.0)


def _node0_stage(h, parts, scale, W1, b1_row, W2, b2_row):
  grid = N // _BN
  return pl.pallas_call(
      _node0_body,
      grid=(grid,),
      in_specs=[
          pl.BlockSpec((_BN, H), lambda i: (i, 0)),
          pl.BlockSpec((1, _BN, H), lambda i: (0, i, 0)),
          pl.BlockSpec((1, _BN, H), lambda i: (1, i, 0)),
          pl.BlockSpec(memory_space=pltpu.SMEM),
          pl.BlockSpec((H, EMB), lambda i: (0, 0)),
          pl.BlockSpec((1, EMB), lambda i: (0, 0)),
          pl.BlockSpec((EMB, EMB), lambda i: (0, 0)),
          pl.BlockSpec((1, EMB), lambda i: (0, 0)),
      ],
      out_specs=pl.BlockSpec((_BN, EMB), lambda i: (i, 0)),
      out_shape=jax.ShapeDtypeStruct((N, EMB), F32),
  )(h, parts, parts, scale, W1, b1_row, W2, b2_row)


def _node1_body(h_ref, p0_ref, p1_ref, sc_ref, w1_ref, b1_ref, w2_ref,
                b2_ref, ow_ref, ob_ref, out_ref):
  hv = sc_ref[0, 0] * h_ref[...] + p0_ref[0] + p1_ref[0]
  t = jnp.maximum(jnp.dot(hv, w1_ref[...], preferred_element_type=F32)
                  + b1_ref[...], 0.0)
  t = jnp.maximum(jnp.dot(t, w2_ref[...], preferred_element_type=F32)
                  + b2_ref[...], 0.0)
  out_ref[...] = jnp.dot(t, ow_ref[...], preferred_element_type=F32) + ob_ref[...]


def _node1_stage(h1, parts, scale, W1, b1_row, W2, b2_row, out_W, ob_row):
  grid = N // _BN
  return pl.pallas_call(
      _node1_body,
      grid=(grid,),
      in_specs=[
          pl.BlockSpec((_BN, EMB), lambda i: (i, 0)),
          pl.BlockSpec((1, _BN, EMB), lambda i: (0, i, 0)),
          pl.BlockSpec((1, _BN, EMB), lambda i: (1, i, 0)),
          pl.BlockSpec(memory_space=pltpu.SMEM),
          pl.BlockSpec((EMB, EMB), lambda i: (0, 0)),
          pl.BlockSpec((1, EMB), lambda i: (0, 0)),
          pl.BlockSpec((EMB, EMB), lambda i: (0, 0)),
          pl.BlockSpec((1, EMB), lambda i: (0, 0)),
          pl.BlockSpec((EMB, PE), lambda i: (0, 0)),
          pl.BlockSpec((1, PE), lambda i: (0, 0)),
      ],
      out_specs=pl.BlockSpec((_BN, PE), lambda i: (i, 0)),
      out_shape=jax.ShapeDtypeStruct((N, PE), F32),
  )(h1, parts, parts, scale, W1, b1_row, W2, b2_row, out_W, ob_row)


def kernel(x, edge_index, edge_attr, eigvecs, eigvals, eps_param, phi_W1,
           phi_b1, phi_W2, phi_b2, edge_W, edge_b, g0_We, g0_be, g0_W1,
           g0_b1, g0_W2, g0_b2, g0_eps, g1_We, g1_be, g1_W1, g1_b1, g1_W2,
           g1_b2, g1_eps, out_W, out_b):
  src = edge_index[0]
  dst = edge_index[1]

  C0 = edge_W @ g0_We
  c0 = edge_b @ g0_We + g0_be
  C1 = edge_W @ g1_We
  c1 = edge_b @ g1_We + g1_be

  h = _eigen_stage(x, eigvecs, eigvals, eps_param.reshape(1, NV), phi_W1,
                   phi_b1.reshape(1, HID), phi_W2, phi_b2.reshape(1, H))
  e0, e1 = _edge_stage(edge_attr, C0, c0.reshape(1, H), C1,
                       c1.reshape(1, EMB))

  parts0 = _sc_segment(H)(h, e0, src, dst)
  h1 = _node0_stage(h, parts0, (1.0 + g0_eps).reshape(1, 1), g0_W1,
                    g0_b1.reshape(1, EMB), g0_W2, g0_b2.reshape(1, EMB))

  parts1 = _sc_segment(EMB)(h1, e1, src, dst)
  pe = _node1_stage(h1, parts1, (1.0 + g1_eps).reshape(1, 1), g1_W1,
                    g1_b1.reshape(1, EMB), g1_W2, g1_b2.reshape(1, EMB),
                    out_W, out_b.reshape(1, PE))
  return pe


# speedup vs baseline: 3.0027x; 1.1185x over previous
import functools

import jax
import jax.numpy as jnp
from jax import lax
from jax.experimental import pallas as pl
from jax.experimental.pallas import tpu as pltpu
from jax.experimental.pallas import tpu_sc as plsc

N = 10000
E = 320000
H = 128
NV = 16
DE = 16
EMB = 64
PE = 32
HID = 2 * H

F32 = jnp.float32


_BN = 1000


def _eigen_body(x_ref, vec_ref, val_ref, eps_ref, w1_ref, b1_ref, w2_ref,
                b2_ref, out_ref):
  ev = val_ref[...] + eps_ref[...]
  ev = jnp.where(jnp.isnan(ev), 0.0, ev)
  vec = jnp.where(jnp.isnan(vec_ref[...]), 0.0, vec_ref[...])
  w1 = w1_ref[...]
  b1 = b1_ref[...]
  w2 = w2_ref[...]
  b2 = b2_ref[...]
  acc = jnp.zeros((_BN, H), F32)
  for v in range(NV):
    t1 = vec[:, v:v + 1] * w1[0:1, :] + ev[:, v:v + 1] * w1[1:2, :] + b1
    t1 = jnp.maximum(t1, 0.0)
    t2 = jnp.dot(t1, w2, preferred_element_type=F32) + b2
    acc = acc + jnp.maximum(t2, 0.0)
  res = x_ref[...] + acc
  out_ref[0] = res[:, :H // 2]
  out_ref[1] = res[:, H // 2:]


def _eigen_stage(x, eigvecs, eigvals, eps_row, phi_W1, phi_b1, phi_W2,
                 phi_b2):
  grid = N // _BN
  return pl.pallas_call(
      _eigen_body,
      grid=(grid,),
      in_specs=[
          pl.BlockSpec((_BN, H), lambda i: (i, 0)),
          pl.BlockSpec((_BN, NV), lambda i: (i, 0)),
          pl.BlockSpec((_BN, NV), lambda i: (i, 0)),
          pl.BlockSpec((1, NV), lambda i: (0, 0)),
          pl.BlockSpec((2, HID), lambda i: (0, 0)),
          pl.BlockSpec((1, HID), lambda i: (0, 0)),
          pl.BlockSpec((HID, H), lambda i: (0, 0)),
          pl.BlockSpec((1, H), lambda i: (0, 0)),
      ],
      out_specs=pl.BlockSpec((2, _BN, H // 2), lambda i: (0, i, 0)),
      out_shape=jax.ShapeDtypeStruct((2, N, H // 2), F32),
  )(x, eigvecs, eigvals, eps_row, phi_W1, phi_b1, phi_W2, phi_b2)


_BE = 4000


def _edge_body(ea_ref, C0_ref, c0_ref, C1_ref, c1_ref, e0_ref, e1_ref):
  ea = ea_ref[...]
  e0 = jnp.dot(ea, C0_ref[...], preferred_element_type=F32) + c0_ref[...]
  e1 = jnp.dot(ea, C1_ref[...], preferred_element_type=F32) + c1_ref[...]
  e0_ref[0] = e0[:, :H // 2]
  e0_ref[1] = e0[:, H // 2:]
  e1_ref[0] = e1[:, :EMB // 2]
  e1_ref[1] = e1[:, EMB // 2:]


def _edge_stage(edge_attr, C0, c0_row, C1, c1_row):
  grid = E // _BE
  return pl.pallas_call(
      _edge_body,
      grid=(grid,),
      in_specs=[
          pl.BlockSpec((_BE, DE), lambda i: (i, 0)),
          pl.BlockSpec((DE, H), lambda i: (0, 0)),
          pl.BlockSpec((1, H), lambda i: (0, 0)),
          pl.BlockSpec((DE, EMB), lambda i: (0, 0)),
          pl.BlockSpec((1, EMB), lambda i: (0, 0)),
      ],
      out_specs=[
          pl.BlockSpec((2, _BE, H // 2), lambda i: (0, i, 0)),
          pl.BlockSpec((2, _BE, EMB // 2), lambda i: (0, i, 0)),
      ],
      out_shape=[
          jax.ShapeDtypeStruct((2, E, H // 2), F32),
          jax.ShapeDtypeStruct((2, E, EMB // 2), F32),
      ],
  )(edge_attr, C0, c0_row, C1, c1_row)


_NC = 2
_NS = 16
_G = 80
_Q = 5
_C = _G * _Q
_NP = 10240


def _make_sc_segment(D):
  DH = D // 2
  per_tile = E // _NS
  chunks = per_tile // _C
  rp = _NP // _NS

  mesh = plsc.VectorSubcoreMesh(core_axis_name="c", subcore_axis_name="s",
                                num_cores=_NC, num_subcores=_NS)

  @functools.partial(
      pl.kernel,
      out_type=jax.ShapeDtypeStruct((_NP, D), F32),
      mesh=mesh,
      scratch_types=[
          pltpu.VMEM((2, _Q, _G), jnp.int32),
          pltpu.VMEM((2, _Q, _G), jnp.int32),
          pltpu.VMEM((_C, DH), F32),
          pltpu.VMEM((_C, DH), F32),
          pltpu.VMEM_SHARED((_NP, DH), F32),
          pltpu.SemaphoreType.DMA,
          pltpu.SemaphoreType.DMA,
          pltpu.SemaphoreType.DMA,
          pltpu.SemaphoreType.DMA,
          pltpu.SemaphoreType.DMA,
      ],
      compiler_params=pltpu.CompilerParams(use_tc_tiling_on_sc=False),
  )
  def seg(h_hbm, e_hbm, srcR, dstR, out_hbm, sidx, didx, bufA, bufB, agg,
          semLA, semLB, semGA, semGB, semS):
    c = lax.axis_index("c")
    s = lax.axis_index("s")
    bufs = (bufA, bufB)
    semL = (semLA, semLB)
    semG = (semGA, semGB)
    h_half = h_hbm.at[c]
    e_half = e_hbm.at[c]

    def zrow(i, carry):
      for j in range(DH // 16):
        bufA[i, pl.ds(16 * j, 16)] = jnp.zeros((16,), F32)
      return carry

    lax.fori_loop(0, 128, zrow, 0)
    for q in range(rp // 128):
      pltpu.sync_copy(bufA.at[pl.ds(0, 128)],
                      agg.at[pl.ds(s * rp + q * 128, 128)])
    plsc.subcore_barrier()

    base_row = s * (per_tile // _G)

    def fire_ld(k, p):
      row = base_row + k * _Q
      pltpu.async_copy(srcR.at[pl.ds(row, _Q)], sidx.at[p], semL[p])
      pltpu.async_copy(dstR.at[pl.ds(row, _Q)], didx.at[p], semL[p])
      pltpu.async_copy(e_half.at[pl.ds(row * _G, _C)], bufs[p], semL[p])

    def wait_ld(k, p):
      row = base_row + k * _Q
      pltpu.make_async_copy(srcR.at[pl.ds(row, _Q)], sidx.at[p],
                            semL[p]).wait()
      pltpu.make_async_copy(dstR.at[pl.ds(row, _Q)], didx.at[p],
                            semL[p]).wait()
      pltpu.make_async_copy(e_half.at[pl.ds(row * _G, _C)], bufs[p],
                            semL[p]).wait()

    def fire_gathers(p):
      for q in range(_Q):
        pltpu.async_copy(h_half.at[sidx.at[p, q]],
                         bufs[p].at[pl.ds(q * _G, _G)], semG[p], add=True)

    def drain_gathers(p):
      for q in range(_Q):
        pltpu.make_async_copy(h_half.at[sidx.at[p, q]],
                              bufs[p].at[pl.ds(q * _G, _G)],
                              semG[p]).wait()

    def scatter(p):
      for q in range(_Q):
        pltpu.async_copy(bufs[p].at[pl.ds(q * _G, _G)],
                         agg.at[didx.at[p, q]], semS, add=True)
      for q in range(_Q):
        pltpu.make_async_copy(bufs[p].at[pl.ds(q * _G, _G)],
                              agg.at[didx.at[p, q]], semS).wait()

    def relu(p):
      buf = bufs[p]

      def rrow(i, cc):
        for j in range(DH // 16):
          sl = pl.ds(16 * j, 16)
          buf[i, sl] = jnp.maximum(buf[i, sl], 0.0)
        return cc

      lax.fori_loop(0, _C, rrow, 0)

    def process(k, p):
      drain_gathers(p)
      relu(p)

      @pl.when(k + 1 < chunks)
      def _():
        wait_ld(k + 1, 1 - p)
        fire_gathers(1 - p)

      scatter(p)

      @pl.when(k + 2 < chunks)
      def _():
        fire_ld(k + 2, p)

    fire_ld(0, 0)
    wait_ld(0, 0)
    fire_gathers(0)
    fire_ld(1, 1)

    def pair(i, carry):
      k = 2 * i
      process(k, 0)

      @pl.when(k + 1 < chunks)
      def _():
        process(k + 1, 1)

      return carry

    lax.fori_loop(0, (chunks + 1) // 2, pair, 0)

    plsc.subcore_barrier()
    pltpu.sync_copy(agg.at[pl.ds(s * rp, rp)],
                    out_hbm.at[pl.ds(s * rp, rp), pl.ds(c * DH, DH)])

  return seg


_sc_cache = {}


def _sc_segment(D):
  if D not in _sc_cache:
    _sc_cache[D] = _make_sc_segment(D)
  return _sc_cache[D]


def _node0_body(h_ref, agg_ref, sc_ref, w1_ref, b1_ref, w2_ref, b2_ref,
                out_ref):
  hv = jnp.concatenate([h_ref[0], h_ref[1]], axis=1)
  hv = sc_ref[0, 0] * hv + agg_ref[...]
  t = jnp.maximum(jnp.dot(hv, w1_ref[...], preferred_element_type=F32)
                  + b1_ref[...], 0.0)
  res = jnp.maximum(
      jnp.dot(t, w2_ref[...], preferred_element_type=F32) + b2_ref[...], 0.0)
  out_ref[0] = res[:, :EMB // 2]
  out_ref[1] = res[:, EMB // 2:]


def _node0_stage(hS, agg, scale, W1, b1_row, W2, b2_row):
  grid = N // _BN
  return pl.pallas_call(
      _node0_body,
      grid=(grid,),
      in_specs=[
          pl.BlockSpec((2, _BN, H // 2), lambda i: (0, i, 0)),
          pl.BlockSpec((_BN, H), lambda i: (i, 0)),
          pl.BlockSpec(memory_space=pltpu.SMEM),
          pl.BlockSpec((H, EMB), lambda i: (0, 0)),
          pl.BlockSpec((1, EMB), lambda i: (0, 0)),
          pl.BlockSpec((EMB, EMB), lambda i: (0, 0)),
          pl.BlockSpec((1, EMB), lambda i: (0, 0)),
      ],
      out_specs=pl.BlockSpec((2, _BN, EMB // 2), lambda i: (0, i, 0)),
      out_shape=jax.ShapeDtypeStruct((2, N, EMB // 2), F32),
  )(hS, agg, scale, W1, b1_row, W2, b2_row)


def _node1_body(h_ref, agg_ref, sc_ref, w1_ref, b1_ref, w2_ref, b2_ref,
                ow_ref, ob_ref, out_ref):
  hv = jnp.concatenate([h_ref[0], h_ref[1]], axis=1)
  hv = sc_ref[0, 0] * hv + agg_ref[...]
  t = jnp.maximum(jnp.dot(hv, w1_ref[...], preferred_element_type=F32)
                  + b1_ref[...], 0.0)
  t = jnp.maximum(jnp.dot(t, w2_ref[...], preferred_element_type=F32)
                  + b2_ref[...], 0.0)
  out_ref[...] = jnp.dot(t, ow_ref[...], preferred_element_type=F32) + ob_ref[...]


def _node1_stage(h1S, agg, scale, W1, b1_row, W2, b2_row, out_W, ob_row):
  grid = N // _BN
  return pl.pallas_call(
      _node1_body,
      grid=(grid,),
      in_specs=[
          pl.BlockSpec((2, _BN, EMB // 2), lambda i: (0, i, 0)),
          pl.BlockSpec((_BN, EMB), lambda i: (i, 0)),
          pl.BlockSpec(memory_space=pltpu.SMEM),
          pl.BlockSpec((EMB, EMB), lambda i: (0, 0)),
          pl.BlockSpec((1, EMB), lambda i: (0, 0)),
          pl.BlockSpec((EMB, EMB), lambda i: (0, 0)),
          pl.BlockSpec((1, EMB), lambda i: (0, 0)),
          pl.BlockSpec((EMB, PE), lambda i: (0, 0)),
          pl.BlockSpec((1, PE), lambda i: (0, 0)),
      ],
      out_specs=pl.BlockSpec((_BN, PE), lambda i: (i, 0)),
      out_shape=jax.ShapeDtypeStruct((N, PE), F32),
  )(h1S, agg, scale, W1, b1_row, W2, b2_row, out_W, ob_row)


def kernel(x, edge_index, edge_attr, eigvecs, eigvals, eps_param, phi_W1,
           phi_b1, phi_W2, phi_b2, edge_W, edge_b, g0_We, g0_be, g0_W1,
           g0_b1, g0_W2, g0_b2, g0_eps, g1_We, g1_be, g1_W1, g1_b1, g1_W2,
           g1_b2, g1_eps, out_W, out_b):
  srcR = edge_index[0].reshape(E // _G, _G)
  dstR = edge_index[1].reshape(E // _G, _G)

  C0 = edge_W @ g0_We
  c0 = edge_b @ g0_We + g0_be
  C1 = edge_W @ g1_We
  c1 = edge_b @ g1_We + g1_be

  hS = _eigen_stage(x, eigvecs, eigvals, eps_param.reshape(1, NV), phi_W1,
                    phi_b1.reshape(1, HID), phi_W2, phi_b2.reshape(1, H))
  e0S, e1S = _edge_stage(edge_attr, C0, c0.reshape(1, H), C1,
                         c1.reshape(1, EMB))

  agg0 = _sc_segment(H)(hS, e0S, srcR, dstR)
  h1S = _node0_stage(hS, agg0, (1.0 + g0_eps).reshape(1, 1), g0_W1,
                     g0_b1.reshape(1, EMB), g0_W2, g0_b2.reshape(1, EMB))

  agg1 = _sc_segment(EMB)(h1S, e1S, srcR, dstR)
  pe = _node1_stage(h1S, agg1, (1.0 + g1_eps).reshape(1, 1), g1_W1,
                    g1_b1.reshape(1, EMB), g1_W2, g1_b2.reshape(1, EMB),
                    out_W, out_b.reshape(1, PE))
  return pe
